# baseline (speedup 1.0000x reference)
import dataclasses
import functools

import jax
import jax.numpy as jnp
from jax import lax
from jax.experimental import pallas as pl
from jax.experimental.pallas import tpu as pltpu
from jax.experimental.pallas import tpu_sc as plsc

BM = 256
NJ = 8
GW = 32
CW = 32


def _routing(topk_ids, N, K, E, NB):
    NK = N * K
    ids = topk_ids.reshape(NK).astype(jnp.int32)
    onehot = (ids[:, None] == jnp.arange(E, dtype=jnp.int32)[None, :]).astype(
        jnp.int32)
    csum = jnp.cumsum(onehot, axis=0)
    counts = csum[-1]
    rank = jnp.take_along_axis(csum, ids[:, None], 1)[:, 0] - 1
    blocks_e = (counts + BM - 1) // BM
    bends = jnp.cumsum(blocks_e)
    bstart = bends - blocks_e
    max_used_e = jnp.max(jnp.where(counts > 0, jnp.arange(E), 0))
    block_expert = jnp.minimum(
        jnp.searchsorted(bends, jnp.arange(NB, dtype=jnp.int32), side="right"),
        max_used_e).astype(jnp.int32)
    pp = (bstart[ids] * BM + rank).astype(jnp.int32)
    eids_plus = jnp.concatenate([block_expert, bends[-1:]])
    return eids_plus, pp


def _sc_dispatch(hidden_states, pp, N, K, P, D):
    info = plsc.get_sparse_core_info()
    NC, NS = info.num_cores, info.num_subcores
    NW = NC * NS
    NK = N * K
    per_w = NK // NW
    tok = (jnp.arange(NK, dtype=jnp.int32) // K).astype(jnp.int32)
    mesh = plsc.VectorSubcoreMesh(core_axis_name="c", subcore_axis_name="s")

    @functools.partial(
        pl.kernel, mesh=mesh,
        out_type=jax.ShapeDtypeStruct((P, D), jnp.float32),
        scratch_types=[pltpu.VMEM((GW,), jnp.int32),
                       pltpu.VMEM((GW,), jnp.int32),
                       pltpu.VMEM((GW, D), jnp.float32),
                       pltpu.SemaphoreType.DMA])
    def dispatch_kernel(hs_hbm, tok_hbm, pp_hbm, gx_hbm,
                        tok_v, pp_v, rows_v, sem):
        wid = lax.axis_index("s") * NC + lax.axis_index("c")
        base = wid * per_w

        @pl.loop(0, per_w, step=GW)
        def _(c):
            off = base + c
            pltpu.sync_copy(tok_hbm.at[pl.ds(off, GW)], tok_v)
            pltpu.sync_copy(pp_hbm.at[pl.ds(off, GW)], pp_v)
            pltpu.async_copy(hs_hbm.at[tok_v], rows_v, sem).wait()
            pltpu.sync_copy(rows_v, gx_hbm.at[pp_v])

    return dispatch_kernel(hidden_states, tok, pp)


def _tc_grouped_mlp(block_expert, gx, w_up, w_down, N, D, H, E, NB, P,
                    interpret=False):
    HT = H // NJ

    def body(eids_ref, wg_ref, wl_ref, wd_ref, gx_ref, o_ref,
             gx_s, wg_s, wl_s, wd_s):
        j = pl.program_id(0)
        b = pl.program_id(1)
        rows = pl.ds(b * BM, BM)

        @pl.when(j == 0)
        def _():
            gx_s[rows, :] = gx_ref[...].astype(jnp.bfloat16)

        valid = jnp.logical_and(j > 0, b < eids_ref[NB])

        @pl.when(valid)
        def _():
            new_tile = jnp.logical_or(
                b == 0, eids_ref[b] != eids_ref[jnp.maximum(b - 1, 0)])

            @pl.when(new_tile)
            def _():
                wg_s[...] = wg_ref[0].astype(jnp.bfloat16)
                wl_s[...] = wl_ref[0].astype(jnp.bfloat16)
                wd_s[...] = wd_ref[0].astype(jnp.bfloat16)

            xb = gx_s[rows, :]
            hg = lax.dot_general(xb, wg_s[...], (((1,), (1,)), ((), ())),
                                 preferred_element_type=jnp.float32)
            hl = lax.dot_general(xb, wl_s[...], (((1,), (1,)), ((), ())),
                                 preferred_element_type=jnp.float32)
            g = (hg * jax.nn.sigmoid(hg)) * hl
            part = lax.dot_general(g.astype(jnp.bfloat16), wd_s[...],
                                   (((1,), (1,)), ((), ())),
                                   preferred_element_type=jnp.float32)

            @pl.when(j == 1)
            def _():
                o_ref[rows, :] = part

            @pl.when(j > 1)
            def _():
                o_ref[rows, :] = o_ref[rows, :] + part

    def _wtile(j, jj):
        return jnp.where(j == 0, 0, jj)

    grid_spec = pltpu.PrefetchScalarGridSpec(
        num_scalar_prefetch=1,
        grid=(NJ + 1, NB),
        in_specs=[
            pl.BlockSpec(
                (1, HT, D),
                lambda j, b, eids: (jnp.where(j == 0, 0, eids[b]),
                                    _wtile(j, jnp.maximum(j - 1, 0)), 0)),
            pl.BlockSpec(
                (1, HT, D),
                lambda j, b, eids: (jnp.where(j == 0, 0, eids[b]),
                                    _wtile(j, NJ + jnp.maximum(j - 1, 0)), 0)),
            pl.BlockSpec(
                (1, D, HT),
                lambda j, b, eids: (jnp.where(j == 0, 0, eids[b]), 0,
                                    _wtile(j, jnp.maximum(j - 1, 0)))),
            pl.BlockSpec((BM, D),
                         lambda j, b, eids: (jnp.where(j == 0, b, 0), 0)),
        ],
        out_specs=pl.BlockSpec((P, D), lambda j, b, eids: (0, 0)),
        scratch_shapes=[pltpu.VMEM((P, D), jnp.bfloat16),
                        pltpu.VMEM((HT, D), jnp.bfloat16),
                        pltpu.VMEM((HT, D), jnp.bfloat16),
                        pltpu.VMEM((D, HT), jnp.bfloat16)],
    )
    return pl.pallas_call(
        body,
        grid_spec=grid_spec,
        out_shape=jax.ShapeDtypeStruct((P, D), jnp.float32),
        compiler_params=pltpu.CompilerParams(
            dimension_semantics=("arbitrary", "arbitrary"),
            vmem_limit_bytes=100 * 1024 * 1024,
        ),
        interpret=interpret,
    )(block_expert, w_up, w_up, w_down, gx)


def _sc_combine(y, topk_weights, ppN, N, K, D):
    info = plsc.get_sparse_core_info()
    NC, NS = info.num_cores, info.num_subcores
    NW = NC * NS
    per_w = N // NW
    mesh = plsc.VectorSubcoreMesh(core_axis_name="c", subcore_axis_name="s")
    idx0 = ppN[:, 0]
    idx1 = ppN[:, 1]
    twf = topk_weights.reshape(N * K).astype(jnp.float32)

    cp = pltpu.CompilerParams()
    if "needs_layout_passes" in pltpu.CompilerParams.__dataclass_fields__:
        cp = dataclasses.replace(cp, needs_layout_passes=False)

    @functools.partial(
        pl.kernel, mesh=mesh,
        out_type=jax.ShapeDtypeStruct((N, D), jnp.float32),
        compiler_params=cp,
        scratch_types=[pltpu.VMEM((CW,), jnp.int32),
                       pltpu.VMEM((CW,), jnp.int32),
                       pltpu.VMEM((CW * 2,), jnp.float32),
                       pltpu.VMEM((CW, D), jnp.float32),
                       pltpu.VMEM((CW, D), jnp.float32),
                       pltpu.SemaphoreType.DMA])
    def combine_kernel(y_hbm, tw_hbm, i0_hbm, i1_hbm, out_hbm,
                       i0_v, i1_v, tw_v, acc_v, rows_v, sem):
        wid = lax.axis_index("s") * NC + lax.axis_index("c")
        base = wid * per_w

        @pl.loop(0, per_w, step=CW)
        def _(c):
            off = base + c
            pltpu.sync_copy(i0_hbm.at[pl.ds(off, CW)], i0_v)
            pltpu.sync_copy(i1_hbm.at[pl.ds(off, CW)], i1_v)
            pltpu.sync_copy(tw_hbm.at[pl.ds(2 * off, 2 * CW)], tw_v)
            cp0 = pltpu.async_copy(y_hbm.at[i0_v], acc_v, sem)
            cp1 = pltpu.async_copy(y_hbm.at[i1_v], rows_v, sem)
            cp0.wait()
            cp1.wait()

            @pl.loop(0, CW)
            def _(t):
                tw0 = plsc.load_gather(
                    tw_v, [jnp.full((16,), 2 * t, jnp.int32)])
                tw1 = plsc.load_gather(
                    tw_v, [jnp.full((16,), 2 * t + 1, jnp.int32)])

                @pl.loop(0, D, step=64)
                def _(d):
                    for u in range(4):
                        sl = pl.ds(d + 16 * u, 16)
                        acc_v[t, sl] = (tw0 * acc_v[t, sl]
                                        + tw1 * rows_v[t, sl])

            pltpu.sync_copy(acc_v, out_hbm.at[pl.ds(off, CW)])

    return combine_kernel(y, twf, idx0, idx1)


def kernel(hidden_states, topk_weights, topk_ids, w_up, w_down):
    N, D = hidden_states.shape
    K = topk_ids.shape[1]
    E = w_up.shape[0]
    H = w_down.shape[2]
    NB = (N * K) // BM + E
    P = NB * BM

    block_expert, pp = _routing(topk_ids, N, K, E, NB)
    gx = _sc_dispatch(hidden_states, pp, N, K, P, D)
    y = _tc_grouped_mlp(block_expert, gx, w_up, w_down, N, D, H, E, NB, P)
    return _sc_combine(y, topk_weights, pp.reshape(N, K), N, K, D)


# speedup vs baseline: 1.2196x; 1.2196x over previous
import dataclasses
import functools

import jax
import jax.numpy as jnp
from jax import lax
from jax.experimental import pallas as pl
from jax.experimental.pallas import tpu as pltpu
from jax.experimental.pallas import tpu_sc as plsc

BM = 256
NJ = 4
GW = 32
CW = 32


def _routing(topk_ids, N, K, E, NB):
    NK = N * K
    ids = topk_ids.reshape(NK).astype(jnp.int32)
    onehot = (ids[:, None] == jnp.arange(E, dtype=jnp.int32)[None, :]).astype(
        jnp.int32)
    csum = jnp.cumsum(onehot, axis=0)
    counts = csum[-1]
    rank = jnp.take_along_axis(csum, ids[:, None], 1)[:, 0] - 1
    blocks_e = (counts + BM - 1) // BM
    bends = jnp.cumsum(blocks_e)
    bstart = bends - blocks_e
    max_used_e = jnp.max(jnp.where(counts > 0, jnp.arange(E), 0))
    block_expert = jnp.minimum(
        jnp.searchsorted(bends, jnp.arange(NB, dtype=jnp.int32), side="right"),
        max_used_e).astype(jnp.int32)
    pp = (bstart[ids] * BM + rank).astype(jnp.int32)
    eids_plus = jnp.concatenate([block_expert, bends[-1:]])
    return eids_plus, pp


def _sc_dispatch(hidden_states, pp, N, K, P, D):
    info = plsc.get_sparse_core_info()
    NC, NS = info.num_cores, info.num_subcores
    NW = NC * NS
    NK = N * K
    per_w = NK // NW
    tok = (jnp.arange(NK, dtype=jnp.int32) // K).astype(jnp.int32)
    mesh = plsc.VectorSubcoreMesh(core_axis_name="c", subcore_axis_name="s")

    @functools.partial(
        pl.kernel, mesh=mesh,
        out_type=jax.ShapeDtypeStruct((P, D), jnp.float32),
        scratch_types=[pltpu.VMEM((GW,), jnp.int32),
                       pltpu.VMEM((GW,), jnp.int32),
                       pltpu.VMEM((GW, D), jnp.float32),
                       pltpu.SemaphoreType.DMA])
    def dispatch_kernel(hs_hbm, tok_hbm, pp_hbm, gx_hbm,
                        tok_v, pp_v, rows_v, sem):
        wid = lax.axis_index("s") * NC + lax.axis_index("c")
        base = wid * per_w

        @pl.loop(0, per_w, step=GW)
        def _(c):
            off = base + c
            pltpu.sync_copy(tok_hbm.at[pl.ds(off, GW)], tok_v)
            pltpu.sync_copy(pp_hbm.at[pl.ds(off, GW)], pp_v)
            pltpu.async_copy(hs_hbm.at[tok_v], rows_v, sem).wait()
            pltpu.sync_copy(rows_v, gx_hbm.at[pp_v])

    return dispatch_kernel(hidden_states, tok, pp)


def _tc_grouped_mlp(block_expert, gx, w_up, w_down, N, D, H, E, NB, P,
                    interpret=False):
    HT = H // NJ

    def body(eids_ref, wg_ref, wl_ref, wd_ref, gx_ref, o_ref,
             gx_s, acc_s, wg_s, wl_s, wd_s):
        j = pl.program_id(0)
        b = pl.program_id(1)
        rows = pl.ds(b * BM, BM)

        @pl.when(j == 0)
        def _():
            gx_s[rows, :] = gx_ref[...].astype(jnp.bfloat16)

        valid = jnp.logical_and(j > 0, b < eids_ref[NB])

        @pl.when(valid)
        def _():
            new_tile = jnp.logical_or(
                b == 0, eids_ref[b] != eids_ref[jnp.maximum(b - 1, 0)])

            @pl.when(new_tile)
            def _():
                wg_s[...] = wg_ref[0].astype(jnp.bfloat16)
                wl_s[...] = wl_ref[0].astype(jnp.bfloat16)
                wd_s[...] = wd_ref[0].astype(jnp.bfloat16)

            xb = gx_s[rows, :]
            hg = lax.dot_general(xb, wg_s[...], (((1,), (1,)), ((), ())),
                                 preferred_element_type=jnp.float32)
            hl = lax.dot_general(xb, wl_s[...], (((1,), (1,)), ((), ())),
                                 preferred_element_type=jnp.float32)
            g = (hg * jax.nn.sigmoid(hg)) * hl
            part = lax.dot_general(g.astype(jnp.bfloat16), wd_s[...],
                                   (((1,), (1,)), ((), ())),
                                   preferred_element_type=jnp.float32)

            @pl.when(j == 1)
            def _():
                acc_s[rows, :] = part

            @pl.when(jnp.logical_and(j > 1, j < NJ))
            def _():
                acc_s[rows, :] = acc_s[rows, :] + part

            @pl.when(j == NJ)
            def _():
                o_ref[...] = acc_s[rows, :] + part

    def _wtile(j, jj):
        return jnp.where(j == 0, 0, jj)

    grid_spec = pltpu.PrefetchScalarGridSpec(
        num_scalar_prefetch=1,
        grid=(NJ + 1, NB),
        in_specs=[
            pl.BlockSpec(
                (1, HT, D),
                lambda j, b, eids: (jnp.where(j == 0, 0, eids[b]),
                                    _wtile(j, jnp.maximum(j - 1, 0)), 0)),
            pl.BlockSpec(
                (1, HT, D),
                lambda j, b, eids: (jnp.where(j == 0, 0, eids[b]),
                                    _wtile(j, NJ + jnp.maximum(j - 1, 0)), 0)),
            pl.BlockSpec(
                (1, D, HT),
                lambda j, b, eids: (jnp.where(j == 0, 0, eids[b]), 0,
                                    _wtile(j, jnp.maximum(j - 1, 0)))),
            pl.BlockSpec((BM, D),
                         lambda j, b, eids: (jnp.where(j == 0, b, 0), 0)),
        ],
        out_specs=pl.BlockSpec(
            (BM, D), lambda j, b, eids: (jnp.where(j == NJ, b, 0), 0)),
        scratch_shapes=[pltpu.VMEM((P, D), jnp.bfloat16),
                        pltpu.VMEM((P, D), jnp.float32),
                        pltpu.VMEM((HT, D), jnp.bfloat16),
                        pltpu.VMEM((HT, D), jnp.bfloat16),
                        pltpu.VMEM((D, HT), jnp.bfloat16)],
    )
    return pl.pallas_call(
        body,
        grid_spec=grid_spec,
        out_shape=jax.ShapeDtypeStruct((P, D), jnp.float32),
        compiler_params=pltpu.CompilerParams(
            dimension_semantics=("arbitrary", "arbitrary"),
            vmem_limit_bytes=100 * 1024 * 1024,
        ),
        interpret=interpret,
    )(block_expert, w_up, w_up, w_down, gx)


def _sc_combine(y, topk_weights, ppN, N, K, D):
    info = plsc.get_sparse_core_info()
    NC, NS = info.num_cores, info.num_subcores
    NW = NC * NS
    per_w = N // NW
    mesh = plsc.VectorSubcoreMesh(core_axis_name="c", subcore_axis_name="s")
    idx0 = ppN[:, 0]
    idx1 = ppN[:, 1]
    twf = topk_weights.reshape(N * K).astype(jnp.float32)

    cp = pltpu.CompilerParams()
    if "needs_layout_passes" in pltpu.CompilerParams.__dataclass_fields__:
        cp = dataclasses.replace(cp, needs_layout_passes=False)

    @functools.partial(
        pl.kernel, mesh=mesh,
        out_type=jax.ShapeDtypeStruct((N, D), jnp.float32),
        compiler_params=cp,
        scratch_types=[pltpu.VMEM((CW,), jnp.int32),
                       pltpu.VMEM((CW,), jnp.int32),
                       pltpu.VMEM((CW * 2,), jnp.float32),
                       pltpu.VMEM((CW, D), jnp.float32),
                       pltpu.VMEM((CW, D), jnp.float32),
                       pltpu.SemaphoreType.DMA])
    def combine_kernel(y_hbm, tw_hbm, i0_hbm, i1_hbm, out_hbm,
                       i0_v, i1_v, tw_v, acc_v, rows_v, sem):
        wid = lax.axis_index("s") * NC + lax.axis_index("c")
        base = wid * per_w

        @pl.loop(0, per_w, step=CW)
        def _(c):
            off = base + c
            pltpu.sync_copy(i0_hbm.at[pl.ds(off, CW)], i0_v)
            pltpu.sync_copy(i1_hbm.at[pl.ds(off, CW)], i1_v)
            pltpu.sync_copy(tw_hbm.at[pl.ds(2 * off, 2 * CW)], tw_v)
            cp0 = pltpu.async_copy(y_hbm.at[i0_v], acc_v, sem)
            cp1 = pltpu.async_copy(y_hbm.at[i1_v], rows_v, sem)
            cp0.wait()
            cp1.wait()

            @pl.loop(0, CW)
            def _(t):
                tw0 = plsc.load_gather(
                    tw_v, [jnp.full((16,), 2 * t, jnp.int32)])
                tw1 = plsc.load_gather(
                    tw_v, [jnp.full((16,), 2 * t + 1, jnp.int32)])

                @pl.loop(0, D, step=64)
                def _(d):
                    for u in range(4):
                        sl = pl.ds(d + 16 * u, 16)
                        acc_v[t, sl] = (tw0 * acc_v[t, sl]
                                        + tw1 * rows_v[t, sl])

            pltpu.sync_copy(acc_v, out_hbm.at[pl.ds(off, CW)])

    return combine_kernel(y, twf, idx0, idx1)


def kernel(hidden_states, topk_weights, topk_ids, w_up, w_down):
    N, D = hidden_states.shape
    K = topk_ids.shape[1]
    E = w_up.shape[0]
    H = w_down.shape[2]
    NB = (N * K) // BM + E
    P = NB * BM

    block_expert, pp = _routing(topk_ids, N, K, E, NB)
    gx = _sc_dispatch(hidden_states, pp, N, K, P, D)
    y = _tc_grouped_mlp(block_expert, gx, w_up, w_down, N, D, H, E, NB, P)
    return _sc_combine(y, topk_weights, pp.reshape(N, K), N, K, D)


# speedup vs baseline: 1.2895x; 1.0573x over previous
import dataclasses
import functools

import jax
import jax.numpy as jnp
from jax import lax
from jax.experimental import pallas as pl
from jax.experimental.pallas import tpu as pltpu
from jax.experimental.pallas import tpu_sc as plsc

BM = 256
NJ = 4
GW = 32
CW = 16


def _routing(topk_ids, N, K, E, NB):
    NK = N * K
    ids = topk_ids.reshape(NK).astype(jnp.int32)
    onehot = (ids[:, None] == jnp.arange(E, dtype=jnp.int32)[None, :]).astype(
        jnp.int32)
    csum = jnp.cumsum(onehot, axis=0)
    counts = csum[-1]
    rank = jnp.sum(onehot * csum, axis=1) - 1
    blocks_e = (counts + BM - 1) // BM
    bends = jnp.cumsum(blocks_e)
    bstart = bends - blocks_e
    max_used_e = jnp.max(jnp.where(counts > 0, jnp.arange(E), 0))
    block_expert = jnp.minimum(
        jnp.sum((jnp.arange(NB, dtype=jnp.int32)[:, None]
                 >= bends[None, :]).astype(jnp.int32), axis=1),
        max_used_e).astype(jnp.int32)
    pp = (jnp.sum(onehot * bstart[None, :], axis=1) * BM
          + rank).astype(jnp.int32)
    eids_plus = jnp.concatenate([block_expert, bends[-1:]])
    return eids_plus, pp


def _sc_dispatch(hidden_states, pp, N, K, P, D):
    info = plsc.get_sparse_core_info()
    NC, NS = info.num_cores, info.num_subcores
    NW = NC * NS
    NK = N * K
    per_w = NK // NW
    tok = (jnp.arange(NK, dtype=jnp.int32) // K).astype(jnp.int32)
    mesh = plsc.VectorSubcoreMesh(core_axis_name="c", subcore_axis_name="s")

    NCH = per_w // GW

    @functools.partial(
        pl.kernel, mesh=mesh,
        out_type=jax.ShapeDtypeStruct((P, D), jnp.float32),
        scratch_types=[pltpu.VMEM((per_w,), jnp.int32),
                       pltpu.VMEM((GW,), jnp.int32),
                       pltpu.VMEM((GW,), jnp.int32),
                       pltpu.VMEM((GW, D), jnp.float32),
                       pltpu.VMEM((GW, D), jnp.float32),
                       pltpu.SemaphoreType.DMA])
    def dispatch_kernel(hs_hbm, tok_hbm, pp_hbm, gx_hbm,
                        tok_all, pp_v0, pp_v1, rows0, rows1, sem):
        wid = lax.axis_index("s") * NC + lax.axis_index("c")
        base = wid * per_w
        pltpu.sync_copy(tok_hbm.at[pl.ds(base, per_w)], tok_all)
        rows = (rows0, rows1)
        ppv = (pp_v0, pp_v1)
        cps = [None] * NCH
        cps[0] = pltpu.async_copy(
            hs_hbm.at[tok_all.at[pl.ds(0, GW)]], rows0, sem)
        for c in range(NCH):
            pltpu.sync_copy(pp_hbm.at[pl.ds(base + c * GW, GW)], ppv[c % 2])
            cps[c].wait()
            if c + 1 < NCH:
                cps[c + 1] = pltpu.async_copy(
                    hs_hbm.at[tok_all.at[pl.ds((c + 1) * GW, GW)]],
                    rows[(c + 1) % 2], sem)
            pltpu.sync_copy(rows[c % 2], gx_hbm.at[ppv[c % 2]])

    return dispatch_kernel(hidden_states, tok, pp)


def _tc_grouped_mlp(block_expert, gx, w_up, w_down, N, D, H, E, NB, P,
                    interpret=False):
    HT = H // NJ

    def body(eids_ref, wg_ref, wl_ref, wd_ref, gx_ref, o_ref,
             gx_s, acc_s, wg_s, wl_s, wd_s):
        j = pl.program_id(0)
        b = pl.program_id(1)
        rows = pl.ds(b * BM, BM)

        @pl.when(j == 0)
        def _():
            gx_s[rows, :] = gx_ref[...].astype(jnp.bfloat16)

        valid = jnp.logical_and(j > 0, b < eids_ref[NB])

        @pl.when(valid)
        def _():
            new_tile = jnp.logical_or(
                b == 0, eids_ref[b] != eids_ref[jnp.maximum(b - 1, 0)])

            @pl.when(new_tile)
            def _():
                wg_s[...] = wg_ref[0].astype(jnp.bfloat16)
                wl_s[...] = wl_ref[0].astype(jnp.bfloat16)
                wd_s[...] = wd_ref[0].astype(jnp.bfloat16)

            xb = gx_s[rows, :]
            hg = lax.dot_general(xb, wg_s[...], (((1,), (1,)), ((), ())),
                                 preferred_element_type=jnp.float32)
            hl = lax.dot_general(xb, wl_s[...], (((1,), (1,)), ((), ())),
                                 preferred_element_type=jnp.float32)
            g = (hg * jax.nn.sigmoid(hg)) * hl
            part = lax.dot_general(g.astype(jnp.bfloat16), wd_s[...],
                                   (((1,), (1,)), ((), ())),
                                   preferred_element_type=jnp.float32)

            @pl.when(j == 1)
            def _():
                acc_s[rows, :] = part

            @pl.when(jnp.logical_and(j > 1, j < NJ))
            def _():
                acc_s[rows, :] = acc_s[rows, :] + part

            @pl.when(j == NJ)
            def _():
                o_ref[...] = acc_s[rows, :] + part

    def _wtile(j, jj):
        return jnp.where(j == 0, 0, jj)

    grid_spec = pltpu.PrefetchScalarGridSpec(
        num_scalar_prefetch=1,
        grid=(NJ + 1, NB),
        in_specs=[
            pl.BlockSpec(
                (1, HT, D),
                lambda j, b, eids: (jnp.where(j == 0, 0, eids[b]),
                                    _wtile(j, jnp.maximum(j - 1, 0)), 0)),
            pl.BlockSpec(
                (1, HT, D),
                lambda j, b, eids: (jnp.where(j == 0, 0, eids[b]),
                                    _wtile(j, NJ + jnp.maximum(j - 1, 0)), 0)),
            pl.BlockSpec(
                (1, D, HT),
                lambda j, b, eids: (jnp.where(j == 0, 0, eids[b]), 0,
                                    _wtile(j, jnp.maximum(j - 1, 0)))),
            pl.BlockSpec((BM, D),
                         lambda j, b, eids: (jnp.where(j == 0, b, 0), 0)),
        ],
        out_specs=pl.BlockSpec(
            (BM, D), lambda j, b, eids: (jnp.where(j == NJ, b, 0), 0)),
        scratch_shapes=[pltpu.VMEM((P, D), jnp.bfloat16),
                        pltpu.VMEM((P, D), jnp.float32),
                        pltpu.VMEM((HT, D), jnp.bfloat16),
                        pltpu.VMEM((HT, D), jnp.bfloat16),
                        pltpu.VMEM((D, HT), jnp.bfloat16)],
    )
    return pl.pallas_call(
        body,
        grid_spec=grid_spec,
        out_shape=jax.ShapeDtypeStruct((P, D), jnp.float32),
        compiler_params=pltpu.CompilerParams(
            dimension_semantics=("arbitrary", "arbitrary"),
            vmem_limit_bytes=100 * 1024 * 1024,
        ),
        interpret=interpret,
    )(block_expert, w_up, w_up, w_down, gx)


def _sc_combine(y, topk_weights, ppN, N, K, D):
    info = plsc.get_sparse_core_info()
    NC, NS = info.num_cores, info.num_subcores
    NW = NC * NS
    per_w = N // NW
    mesh = plsc.VectorSubcoreMesh(core_axis_name="c", subcore_axis_name="s")
    idx0 = ppN[:, 0]
    idx1 = ppN[:, 1]
    twf = topk_weights.reshape(N * K).astype(jnp.float32)

    cp = pltpu.CompilerParams()
    if "needs_layout_passes" in pltpu.CompilerParams.__dataclass_fields__:
        cp = dataclasses.replace(cp, needs_layout_passes=False)

    NCH = per_w // CW

    @functools.partial(
        pl.kernel, mesh=mesh,
        out_type=jax.ShapeDtypeStruct((N, D), jnp.float32),
        compiler_params=cp,
        scratch_types=[pltpu.VMEM((per_w,), jnp.int32),
                       pltpu.VMEM((per_w,), jnp.int32),
                       pltpu.VMEM((per_w * 2,), jnp.float32),
                       pltpu.VMEM((CW, D), jnp.float32),
                       pltpu.VMEM((CW, D), jnp.float32),
                       pltpu.VMEM((CW, D), jnp.float32),
                       pltpu.VMEM((CW, D), jnp.float32),
                       pltpu.SemaphoreType.DMA])
    def combine_kernel(y_hbm, tw_hbm, i0_hbm, i1_hbm, out_hbm,
                       i0_all, i1_all, tw_all, acc0, acc1, rw0, rw1, sem):
        wid = lax.axis_index("s") * NC + lax.axis_index("c")
        base = wid * per_w
        pltpu.sync_copy(i0_hbm.at[pl.ds(base, per_w)], i0_all)
        pltpu.sync_copy(i1_hbm.at[pl.ds(base, per_w)], i1_all)
        pltpu.sync_copy(tw_hbm.at[pl.ds(2 * base, 2 * per_w)], tw_all)
        acc = (acc0, acc1)
        rw = (rw0, rw1)

        def issue(c, buf):
            c0 = pltpu.async_copy(
                y_hbm.at[i0_all.at[pl.ds(c * CW, CW)]], acc[buf], sem)
            c1 = pltpu.async_copy(
                y_hbm.at[i1_all.at[pl.ds(c * CW, CW)]], rw[buf], sem)
            return c0, c1

        cps = [None] * NCH
        cps[0] = issue(0, 0)
        for c in range(NCH):
            cps[c][0].wait()
            cps[c][1].wait()
            if c + 1 < NCH:
                cps[c + 1] = issue(c + 1, (c + 1) % 2)
            av, rv = acc[c % 2], rw[c % 2]

            @pl.loop(0, CW)
            def _(t):
                tw0 = plsc.load_gather(
                    tw_all, [jnp.full((16,), 2 * (c * CW + t), jnp.int32)])
                tw1 = plsc.load_gather(
                    tw_all, [jnp.full((16,), 2 * (c * CW + t) + 1,
                                      jnp.int32)])

                @pl.loop(0, D, step=64)
                def _(d):
                    for u in range(4):
                        sl = pl.ds(d + 16 * u, 16)
                        av[t, sl] = tw0 * av[t, sl] + tw1 * rv[t, sl]

            pltpu.sync_copy(av, out_hbm.at[pl.ds(base + c * CW, CW)])

    return combine_kernel(y, twf, idx0, idx1)


def kernel(hidden_states, topk_weights, topk_ids, w_up, w_down):
    N, D = hidden_states.shape
    K = topk_ids.shape[1]
    E = w_up.shape[0]
    H = w_down.shape[2]
    NB = (N * K) // BM + E
    P = NB * BM

    block_expert, pp = _routing(topk_ids, N, K, E, NB)
    gx = _sc_dispatch(hidden_states, pp, N, K, P, D)
    y = _tc_grouped_mlp(block_expert, gx, w_up, w_down, N, D, H, E, NB, P)
    return _sc_combine(y, topk_weights, pp.reshape(N, K), N, K, D)


# speedup vs baseline: 1.3104x; 1.0162x over previous
import dataclasses
import functools

import jax
import jax.numpy as jnp
from jax import lax
from jax.experimental import pallas as pl
from jax.experimental.pallas import tpu as pltpu
from jax.experimental.pallas import tpu_sc as plsc

BM = 256
NJ = 4
GW = 32
CW = 16


def _routing(topk_ids, N, K, E, NB):
    NK = N * K
    ids = topk_ids.reshape(NK).astype(jnp.int32)
    onehot = (ids[:, None] == jnp.arange(E, dtype=jnp.int32)[None, :]).astype(
        jnp.int32)
    SB = 128
    NSB = NK // SB
    ohb = onehot.astype(jnp.bfloat16).reshape(NSB, SB, E)
    tri = jnp.tril(jnp.ones((SB, SB), jnp.bfloat16))
    inner = jnp.einsum("ij,bjk->bik", tri, ohb,
                       preferred_element_type=jnp.float32)
    sums = inner[:, -1, :]
    tri_x = jnp.tril(jnp.ones((NSB, NSB), jnp.bfloat16), k=-1)
    off = jnp.einsum("ij,jk->ik", tri_x, sums.astype(jnp.bfloat16),
                     preferred_element_type=jnp.float32)
    csum = (inner + off[:, None, :]).reshape(NK, E).astype(jnp.int32)
    counts = csum[-1]
    rank = jnp.sum(onehot * csum, axis=1) - 1
    blocks_e = (counts + BM - 1) // BM
    bends = jnp.cumsum(blocks_e)
    bstart = bends - blocks_e
    max_used_e = jnp.max(jnp.where(counts > 0, jnp.arange(E), 0))
    block_expert = jnp.minimum(
        jnp.sum((jnp.arange(NB, dtype=jnp.int32)[:, None]
                 >= bends[None, :]).astype(jnp.int32), axis=1),
        max_used_e).astype(jnp.int32)
    pp = (jnp.sum(onehot * bstart[None, :], axis=1) * BM
          + rank).astype(jnp.int32)
    eids_plus = jnp.concatenate([block_expert, bends[-1:]])
    return eids_plus, pp


def _sc_dispatch(hidden_states, pp, N, K, P, D):
    info = plsc.get_sparse_core_info()
    NC, NS = info.num_cores, info.num_subcores
    NW = NC * NS
    NK = N * K
    per_w = NK // NW
    tok = (jnp.arange(NK, dtype=jnp.int32) // K).astype(jnp.int32)
    mesh = plsc.VectorSubcoreMesh(core_axis_name="c", subcore_axis_name="s")

    NCH = per_w // GW

    @functools.partial(
        pl.kernel, mesh=mesh,
        out_type=jax.ShapeDtypeStruct((P, D), jnp.float32),
        scratch_types=[pltpu.VMEM((per_w,), jnp.int32),
                       pltpu.VMEM((GW,), jnp.int32),
                       pltpu.VMEM((GW,), jnp.int32),
                       pltpu.VMEM((GW, D), jnp.float32),
                       pltpu.VMEM((GW, D), jnp.float32),
                       pltpu.SemaphoreType.DMA])
    def dispatch_kernel(hs_hbm, tok_hbm, pp_hbm, gx_hbm,
                        tok_all, pp_v0, pp_v1, rows0, rows1, sem):
        wid = lax.axis_index("s") * NC + lax.axis_index("c")
        base = wid * per_w
        pltpu.sync_copy(tok_hbm.at[pl.ds(base, per_w)], tok_all)
        rows = (rows0, rows1)
        ppv = (pp_v0, pp_v1)
        cps = [None] * NCH
        cps[0] = pltpu.async_copy(
            hs_hbm.at[tok_all.at[pl.ds(0, GW)]], rows0, sem)
        for c in range(NCH):
            pltpu.sync_copy(pp_hbm.at[pl.ds(base + c * GW, GW)], ppv[c % 2])
            cps[c].wait()
            if c + 1 < NCH:
                cps[c + 1] = pltpu.async_copy(
                    hs_hbm.at[tok_all.at[pl.ds((c + 1) * GW, GW)]],
                    rows[(c + 1) % 2], sem)
            pltpu.sync_copy(rows[c % 2], gx_hbm.at[ppv[c % 2]])

    return dispatch_kernel(hidden_states, tok, pp)


def _tc_grouped_mlp(block_expert, gx, w_up, w_down, N, D, H, E, NB, P,
                    interpret=False):
    HT = H // NJ

    def body(eids_ref, wg_ref, wl_ref, wd_ref, gx_ref, o_ref,
             gx_s, acc_s, wg_s, wl_s, wd_s):
        j = pl.program_id(0)
        b = pl.program_id(1)
        rows = pl.ds(b * BM, BM)

        @pl.when(j == 0)
        def _():
            gx_s[rows, :] = gx_ref[...].astype(jnp.bfloat16)

        valid = jnp.logical_and(j > 0, b < eids_ref[NB])

        @pl.when(valid)
        def _():
            new_tile = jnp.logical_or(
                b == 0, eids_ref[b] != eids_ref[jnp.maximum(b - 1, 0)])

            @pl.when(new_tile)
            def _():
                wg_s[...] = wg_ref[0].astype(jnp.bfloat16)
                wl_s[...] = wl_ref[0].astype(jnp.bfloat16)
                wd_s[...] = wd_ref[0].astype(jnp.bfloat16)

            xb = gx_s[rows, :]
            hg = lax.dot_general(xb, wg_s[...], (((1,), (1,)), ((), ())),
                                 preferred_element_type=jnp.float32)
            hl = lax.dot_general(xb, wl_s[...], (((1,), (1,)), ((), ())),
                                 preferred_element_type=jnp.float32)
            g = (hg * jax.nn.sigmoid(hg)) * hl
            part = lax.dot_general(g.astype(jnp.bfloat16), wd_s[...],
                                   (((1,), (1,)), ((), ())),
                                   preferred_element_type=jnp.float32)

            @pl.when(j == 1)
            def _():
                acc_s[rows, :] = part

            @pl.when(jnp.logical_and(j > 1, j < NJ))
            def _():
                acc_s[rows, :] = acc_s[rows, :] + part

            @pl.when(j == NJ)
            def _():
                o_ref[...] = acc_s[rows, :] + part

    def _wtile(j, jj):
        return jnp.where(j == 0, 0, jj)

    grid_spec = pltpu.PrefetchScalarGridSpec(
        num_scalar_prefetch=1,
        grid=(NJ + 1, NB),
        in_specs=[
            pl.BlockSpec(
                (1, HT, D),
                lambda j, b, eids: (jnp.where(j == 0, 0, eids[b]),
                                    _wtile(j, jnp.maximum(j - 1, 0)), 0)),
            pl.BlockSpec(
                (1, HT, D),
                lambda j, b, eids: (jnp.where(j == 0, 0, eids[b]),
                                    _wtile(j, NJ + jnp.maximum(j - 1, 0)), 0)),
            pl.BlockSpec(
                (1, D, HT),
                lambda j, b, eids: (jnp.where(j == 0, 0, eids[b]), 0,
                                    _wtile(j, jnp.maximum(j - 1, 0)))),
            pl.BlockSpec((BM, D),
                         lambda j, b, eids: (jnp.where(j == 0, b, 0), 0)),
        ],
        out_specs=pl.BlockSpec(
            (BM, D), lambda j, b, eids: (jnp.where(j == NJ, b, 0), 0)),
        scratch_shapes=[pltpu.VMEM((P, D), jnp.bfloat16),
                        pltpu.VMEM((P, D), jnp.float32),
                        pltpu.VMEM((HT, D), jnp.bfloat16),
                        pltpu.VMEM((HT, D), jnp.bfloat16),
                        pltpu.VMEM((D, HT), jnp.bfloat16)],
    )
    return pl.pallas_call(
        body,
        grid_spec=grid_spec,
        out_shape=jax.ShapeDtypeStruct((P, D), jnp.float32),
        compiler_params=pltpu.CompilerParams(
            dimension_semantics=("arbitrary", "arbitrary"),
            vmem_limit_bytes=100 * 1024 * 1024,
        ),
        interpret=interpret,
    )(block_expert, w_up, w_up, w_down, gx)


def _sc_combine(y, topk_weights, ppN, N, K, D):
    info = plsc.get_sparse_core_info()
    NC, NS = info.num_cores, info.num_subcores
    NW = NC * NS
    per_w = N // NW
    mesh = plsc.VectorSubcoreMesh(core_axis_name="c", subcore_axis_name="s")
    idx0 = ppN[:, 0]
    idx1 = ppN[:, 1]
    twf = topk_weights.reshape(N * K).astype(jnp.float32)

    cp = pltpu.CompilerParams()
    if "needs_layout_passes" in pltpu.CompilerParams.__dataclass_fields__:
        cp = dataclasses.replace(cp, needs_layout_passes=False)

    NCH = per_w // CW

    @functools.partial(
        pl.kernel, mesh=mesh,
        out_type=jax.ShapeDtypeStruct((N, D), jnp.float32),
        compiler_params=cp,
        scratch_types=[pltpu.VMEM((per_w,), jnp.int32),
                       pltpu.VMEM((per_w,), jnp.int32),
                       pltpu.VMEM((per_w * 2,), jnp.float32),
                       pltpu.VMEM((CW, D), jnp.float32),
                       pltpu.VMEM((CW, D), jnp.float32),
                       pltpu.VMEM((CW, D), jnp.float32),
                       pltpu.VMEM((CW, D), jnp.float32),
                       pltpu.SemaphoreType.DMA])
    def combine_kernel(y_hbm, tw_hbm, i0_hbm, i1_hbm, out_hbm,
                       i0_all, i1_all, tw_all, acc0, acc1, rw0, rw1, sem):
        wid = lax.axis_index("s") * NC + lax.axis_index("c")
        base = wid * per_w
        pltpu.sync_copy(i0_hbm.at[pl.ds(base, per_w)], i0_all)
        pltpu.sync_copy(i1_hbm.at[pl.ds(base, per_w)], i1_all)
        pltpu.sync_copy(tw_hbm.at[pl.ds(2 * base, 2 * per_w)], tw_all)
        acc = (acc0, acc1)
        rw = (rw0, rw1)

        def issue(c, buf):
            c0 = pltpu.async_copy(
                y_hbm.at[i0_all.at[pl.ds(c * CW, CW)]], acc[buf], sem)
            c1 = pltpu.async_copy(
                y_hbm.at[i1_all.at[pl.ds(c * CW, CW)]], rw[buf], sem)
            return c0, c1

        cps = [None] * NCH
        cps[0] = issue(0, 0)
        for c in range(NCH):
            cps[c][0].wait()
            cps[c][1].wait()
            if c + 1 < NCH:
                cps[c + 1] = issue(c + 1, (c + 1) % 2)
            av, rv = acc[c % 2], rw[c % 2]

            @pl.loop(0, CW)
            def _(t):
                tw0 = plsc.load_gather(
                    tw_all, [jnp.full((16,), 2 * (c * CW + t), jnp.int32)])
                tw1 = plsc.load_gather(
                    tw_all, [jnp.full((16,), 2 * (c * CW + t) + 1,
                                      jnp.int32)])

                @pl.loop(0, D, step=64)
                def _(d):
                    for u in range(4):
                        sl = pl.ds(d + 16 * u, 16)
                        av[t, sl] = tw0 * av[t, sl] + tw1 * rv[t, sl]

            pltpu.sync_copy(av, out_hbm.at[pl.ds(base + c * CW, CW)])

    return combine_kernel(y, twf, idx0, idx1)


def kernel(hidden_states, topk_weights, topk_ids, w_up, w_down):
    N, D = hidden_states.shape
    K = topk_ids.shape[1]
    E = w_up.shape[0]
    H = w_down.shape[2]
    NB = (N * K) // BM + E
    P = NB * BM

    block_expert, pp = _routing(topk_ids, N, K, E, NB)
    gx = _sc_dispatch(hidden_states, pp, N, K, P, D)
    y = _tc_grouped_mlp(block_expert, gx, w_up, w_down, N, D, H, E, NB, P)
    return _sc_combine(y, topk_weights, pp.reshape(N, K), N, K, D)


# speedup vs baseline: 1.3892x; 1.0602x over previous
import dataclasses
import functools

import jax
import jax.numpy as jnp
from jax import lax
from jax.experimental import pallas as pl
from jax.experimental.pallas import tpu as pltpu
from jax.experimental.pallas import tpu_sc as plsc

BM = 256
NJ = 4
GW = 32
CW = 16


def _routing(topk_ids, N, K, E, NB):
    NK = N * K
    ids = topk_ids.reshape(NK).astype(jnp.int32)
    onehot = (ids[:, None] == jnp.arange(E, dtype=jnp.int32)[None, :]).astype(
        jnp.int32)
    SB = 128
    NSB = NK // SB
    ohb = onehot.astype(jnp.bfloat16).reshape(NSB, SB, E)
    tri = jnp.tril(jnp.ones((SB, SB), jnp.bfloat16))
    inner = jnp.einsum("ij,bjk->bik", tri, ohb,
                       preferred_element_type=jnp.float32)
    sums = inner[:, -1, :]
    tri_x = jnp.tril(jnp.ones((NSB, NSB), jnp.bfloat16), k=-1)
    off = jnp.einsum("ij,jk->ik", tri_x, sums.astype(jnp.bfloat16),
                     preferred_element_type=jnp.float32)
    csum = (inner + off[:, None, :]).reshape(NK, E).astype(jnp.int32)
    counts = csum[-1]
    rank = jnp.sum(onehot * csum, axis=1) - 1
    blocks_e = (counts + BM - 1) // BM
    bends = jnp.cumsum(blocks_e)
    bstart = bends - blocks_e
    max_used_e = jnp.max(jnp.where(counts > 0, jnp.arange(E), 0))
    block_expert = jnp.minimum(
        jnp.sum((jnp.arange(NB, dtype=jnp.int32)[:, None]
                 >= bends[None, :]).astype(jnp.int32), axis=1),
        max_used_e).astype(jnp.int32)
    pp = (jnp.sum(onehot * bstart[None, :], axis=1) * BM
          + rank).astype(jnp.int32)
    eids_plus = jnp.concatenate([block_expert, bends[-1:]])
    return eids_plus, pp


def _sc_dispatch(hidden_states, pp, N, K, P, D):
    info = plsc.get_sparse_core_info()
    NC, NS = info.num_cores, info.num_subcores
    NW = NC * NS
    NK = N * K
    per_w = NK // NW
    tok = (jnp.arange(NK, dtype=jnp.int32) // K).astype(jnp.int32)
    mesh = plsc.VectorSubcoreMesh(core_axis_name="c", subcore_axis_name="s")

    NCH = per_w // GW

    @functools.partial(
        pl.kernel, mesh=mesh,
        out_type=jax.ShapeDtypeStruct((P, D), jnp.float32),
        scratch_types=[pltpu.VMEM((per_w,), jnp.int32),
                       pltpu.VMEM((GW,), jnp.int32),
                       pltpu.VMEM((GW,), jnp.int32),
                       pltpu.VMEM((GW, D), jnp.float32),
                       pltpu.VMEM((GW, D), jnp.float32),
                       pltpu.SemaphoreType.DMA])
    def dispatch_kernel(hs_hbm, tok_hbm, pp_hbm, gx_hbm,
                        tok_all, pp_v0, pp_v1, rows0, rows1, sem):
        wid = lax.axis_index("s") * NC + lax.axis_index("c")
        base = wid * per_w
        pltpu.sync_copy(tok_hbm.at[pl.ds(base, per_w)], tok_all)
        rows = (rows0, rows1)
        ppv = (pp_v0, pp_v1)
        cps = [None] * NCH
        cps[0] = pltpu.async_copy(
            hs_hbm.at[tok_all.at[pl.ds(0, GW)]], rows0, sem)
        for c in range(NCH):
            pltpu.sync_copy(pp_hbm.at[pl.ds(base + c * GW, GW)], ppv[c % 2])
            cps[c].wait()
            if c + 1 < NCH:
                cps[c + 1] = pltpu.async_copy(
                    hs_hbm.at[tok_all.at[pl.ds((c + 1) * GW, GW)]],
                    rows[(c + 1) % 2], sem)
            pltpu.sync_copy(rows[c % 2], gx_hbm.at[ppv[c % 2]])

    return dispatch_kernel(hidden_states, tok, pp)


def _tc_grouped_mlp(block_expert, gx, w_up, w_down, N, D, H, E, NB, P,
                    interpret=False):
    HT = H // NJ

    def body(eids_ref, wg_ref, wl_ref, wd_ref, gx_ref, o_ref,
             gx_s, acc_s, wg_s, wl_s, wd_s):
        j = pl.program_id(0)
        b = pl.program_id(1)
        rows = pl.ds(b * BM, BM)

        @pl.when(b < eids_ref[NB])
        def _():
            new_tile = jnp.logical_or(
                b == 0, eids_ref[b] != eids_ref[jnp.maximum(b - 1, 0)])

            @pl.when(new_tile)
            def _():
                wg_s[...] = wg_ref[0].astype(jnp.bfloat16)
                wl_s[...] = wl_ref[0].astype(jnp.bfloat16)
                wd_s[...] = wd_ref[0].astype(jnp.bfloat16)

            def compute(xb):
                hg = lax.dot_general(xb, wg_s[...], (((1,), (1,)), ((), ())),
                                     preferred_element_type=jnp.float32)
                hl = lax.dot_general(xb, wl_s[...], (((1,), (1,)), ((), ())),
                                     preferred_element_type=jnp.float32)
                g = (hg * jax.nn.sigmoid(hg)) * hl
                return lax.dot_general(g.astype(jnp.bfloat16), wd_s[...],
                                       (((1,), (1,)), ((), ())),
                                       preferred_element_type=jnp.float32)

            @pl.when(j == 0)
            def _():
                xb = gx_ref[...].astype(jnp.bfloat16)
                gx_s[rows, :] = xb
                acc_s[rows, :] = compute(xb).astype(jnp.bfloat16)

            @pl.when(jnp.logical_and(j > 0, j < NJ - 1))
            def _():
                part = compute(gx_s[rows, :])
                acc_s[rows, :] = (acc_s[rows, :].astype(jnp.float32)
                                  + part).astype(jnp.bfloat16)

            @pl.when(j == NJ - 1)
            def _():
                part = compute(gx_s[rows, :])
                o_ref[...] = acc_s[rows, :].astype(jnp.float32) + part

    grid_spec = pltpu.PrefetchScalarGridSpec(
        num_scalar_prefetch=1,
        grid=(NJ, NB),
        in_specs=[
            pl.BlockSpec((1, HT, D), lambda j, b, eids: (eids[b], j, 0)),
            pl.BlockSpec((1, HT, D),
                         lambda j, b, eids: (eids[b], NJ + j, 0)),
            pl.BlockSpec((1, D, HT), lambda j, b, eids: (eids[b], 0, j)),
            pl.BlockSpec((BM, D),
                         lambda j, b, eids: (jnp.where(j == 0, b, 0), 0)),
        ],
        out_specs=pl.BlockSpec(
            (BM, D), lambda j, b, eids: (jnp.where(j == NJ - 1, b, 0), 0)),
        scratch_shapes=[pltpu.VMEM((P, D), jnp.bfloat16),
                        pltpu.VMEM((P, D), jnp.bfloat16),
                        pltpu.VMEM((HT, D), jnp.bfloat16),
                        pltpu.VMEM((HT, D), jnp.bfloat16),
                        pltpu.VMEM((D, HT), jnp.bfloat16)],
    )
    return pl.pallas_call(
        body,
        grid_spec=grid_spec,
        out_shape=jax.ShapeDtypeStruct((P, D), jnp.float32),
        compiler_params=pltpu.CompilerParams(
            dimension_semantics=("arbitrary", "arbitrary"),
            vmem_limit_bytes=100 * 1024 * 1024,
        ),
        interpret=interpret,
    )(block_expert, w_up, w_up, w_down, gx)


def _sc_combine(y, topk_weights, ppN, N, K, D):
    info = plsc.get_sparse_core_info()
    NC, NS = info.num_cores, info.num_subcores
    NW = NC * NS
    per_w = N // NW
    mesh = plsc.VectorSubcoreMesh(core_axis_name="c", subcore_axis_name="s")
    idx0 = ppN[:, 0]
    idx1 = ppN[:, 1]
    twf = topk_weights.reshape(N * K).astype(jnp.float32)

    cp = pltpu.CompilerParams()
    if "needs_layout_passes" in pltpu.CompilerParams.__dataclass_fields__:
        cp = dataclasses.replace(cp, needs_layout_passes=False)

    NCH = per_w // CW

    @functools.partial(
        pl.kernel, mesh=mesh,
        out_type=jax.ShapeDtypeStruct((N, D), jnp.float32),
        compiler_params=cp,
        scratch_types=[pltpu.VMEM((per_w,), jnp.int32),
                       pltpu.VMEM((per_w,), jnp.int32),
                       pltpu.VMEM((per_w * 2,), jnp.float32),
                       pltpu.VMEM((CW, D), jnp.float32),
                       pltpu.VMEM((CW, D), jnp.float32),
                       pltpu.VMEM((CW, D), jnp.float32),
                       pltpu.VMEM((CW, D), jnp.float32),
                       pltpu.SemaphoreType.DMA])
    def combine_kernel(y_hbm, tw_hbm, i0_hbm, i1_hbm, out_hbm,
                       i0_all, i1_all, tw_all, acc0, acc1, rw0, rw1, sem):
        wid = lax.axis_index("s") * NC + lax.axis_index("c")
        base = wid * per_w
        pltpu.sync_copy(i0_hbm.at[pl.ds(base, per_w)], i0_all)
        pltpu.sync_copy(i1_hbm.at[pl.ds(base, per_w)], i1_all)
        pltpu.sync_copy(tw_hbm.at[pl.ds(2 * base, 2 * per_w)], tw_all)
        acc = (acc0, acc1)
        rw = (rw0, rw1)

        def issue(c, buf):
            c0 = pltpu.async_copy(
                y_hbm.at[i0_all.at[pl.ds(c * CW, CW)]], acc[buf], sem)
            c1 = pltpu.async_copy(
                y_hbm.at[i1_all.at[pl.ds(c * CW, CW)]], rw[buf], sem)
            return c0, c1

        cps = [None] * NCH
        cps[0] = issue(0, 0)
        for c in range(NCH):
            cps[c][0].wait()
            cps[c][1].wait()
            if c + 1 < NCH:
                cps[c + 1] = issue(c + 1, (c + 1) % 2)
            av, rv = acc[c % 2], rw[c % 2]

            @pl.loop(0, CW)
            def _(t):
                tw0 = plsc.load_gather(
                    tw_all, [jnp.full((16,), 2 * (c * CW + t), jnp.int32)])
                tw1 = plsc.load_gather(
                    tw_all, [jnp.full((16,), 2 * (c * CW + t) + 1,
                                      jnp.int32)])

                @pl.loop(0, D, step=64)
                def _(d):
                    for u in range(4):
                        sl = pl.ds(d + 16 * u, 16)
                        av[t, sl] = tw0 * av[t, sl] + tw1 * rv[t, sl]

            pltpu.sync_copy(av, out_hbm.at[pl.ds(base + c * CW, CW)])

    return combine_kernel(y, twf, idx0, idx1)


def kernel(hidden_states, topk_weights, topk_ids, w_up, w_down):
    N, D = hidden_states.shape
    K = topk_ids.shape[1]
    E = w_up.shape[0]
    H = w_down.shape[2]
    NB = (N * K) // BM + E
    P = NB * BM

    block_expert, pp = _routing(topk_ids, N, K, E, NB)
    gx = _sc_dispatch(hidden_states, pp, N, K, P, D)
    y = _tc_grouped_mlp(block_expert, gx, w_up, w_down, N, D, H, E, NB, P)
    return _sc_combine(y, topk_weights, pp.reshape(N, K), N, K, D)


# speedup vs baseline: 1.3942x; 1.0036x over previous
import dataclasses
import functools

import jax
import jax.numpy as jnp
from jax import lax
from jax.experimental import pallas as pl
from jax.experimental.pallas import tpu as pltpu
from jax.experimental.pallas import tpu_sc as plsc

BM = 256
NJ = 4
GW = 32
CW = 16


def _routing(topk_ids, N, K, E, NB):
    NK = N * K
    ids = topk_ids.reshape(NK).astype(jnp.int32)
    onehot = (ids[:, None] == jnp.arange(E, dtype=jnp.int32)[None, :]).astype(
        jnp.int32)
    SB = 128
    NSB = NK // SB
    ohb = onehot.astype(jnp.bfloat16).reshape(NSB, SB, E)
    tri = jnp.tril(jnp.ones((SB, SB), jnp.bfloat16))
    inner = jnp.einsum("ij,bjk->bik", tri, ohb,
                       preferred_element_type=jnp.float32)
    sums = inner[:, -1, :]
    tri_x = jnp.tril(jnp.ones((NSB, NSB), jnp.bfloat16), k=-1)
    off = jnp.einsum("ij,jk->ik", tri_x, sums.astype(jnp.bfloat16),
                     preferred_element_type=jnp.float32)
    csum = (inner + off[:, None, :]).reshape(NK, E).astype(jnp.int32)
    counts = csum[-1]
    rank = jnp.sum(onehot * csum, axis=1) - 1
    blocks_e = (counts + BM - 1) // BM
    bends = jnp.dot(jnp.tril(jnp.ones((E, E), jnp.float32)),
                    blocks_e.astype(jnp.float32)).astype(jnp.int32)
    bstart = bends - blocks_e
    max_used_e = jnp.max(jnp.where(counts > 0, jnp.arange(E), 0))
    block_expert = jnp.minimum(
        jnp.sum((jnp.arange(NB, dtype=jnp.int32)[:, None]
                 >= bends[None, :]).astype(jnp.int32), axis=1),
        max_used_e).astype(jnp.int32)
    pp = (jnp.sum(onehot * bstart[None, :], axis=1) * BM
          + rank).astype(jnp.int32)
    eids_plus = jnp.concatenate([block_expert, bends[-1:]])
    return eids_plus, pp


def _sc_dispatch(hidden_states, pp, N, K, P, D):
    info = plsc.get_sparse_core_info()
    NC, NS = info.num_cores, info.num_subcores
    NW = NC * NS
    NK = N * K
    per_w = NK // NW
    tok = (jnp.arange(NK, dtype=jnp.int32) // K).astype(jnp.int32)
    mesh = plsc.VectorSubcoreMesh(core_axis_name="c", subcore_axis_name="s")

    NCH = per_w // GW

    @functools.partial(
        pl.kernel, mesh=mesh,
        out_type=jax.ShapeDtypeStruct((P, D), jnp.float32),
        scratch_types=[pltpu.VMEM((per_w,), jnp.int32),
                       pltpu.VMEM((GW,), jnp.int32),
                       pltpu.VMEM((GW,), jnp.int32),
                       pltpu.VMEM((GW, D), jnp.float32),
                       pltpu.VMEM((GW, D), jnp.float32),
                       pltpu.SemaphoreType.DMA])
    def dispatch_kernel(hs_hbm, tok_hbm, pp_hbm, gx_hbm,
                        tok_all, pp_v0, pp_v1, rows0, rows1, sem):
        wid = lax.axis_index("s") * NC + lax.axis_index("c")
        base = wid * per_w
        pltpu.sync_copy(tok_hbm.at[pl.ds(base, per_w)], tok_all)
        rows = (rows0, rows1)
        ppv = (pp_v0, pp_v1)
        cps = [None] * NCH
        cps[0] = pltpu.async_copy(
            hs_hbm.at[tok_all.at[pl.ds(0, GW)]], rows0, sem)
        for c in range(NCH):
            pltpu.sync_copy(pp_hbm.at[pl.ds(base + c * GW, GW)], ppv[c % 2])
            cps[c].wait()
            if c + 1 < NCH:
                cps[c + 1] = pltpu.async_copy(
                    hs_hbm.at[tok_all.at[pl.ds((c + 1) * GW, GW)]],
                    rows[(c + 1) % 2], sem)
            pltpu.sync_copy(rows[c % 2], gx_hbm.at[ppv[c % 2]])

    return dispatch_kernel(hidden_states, tok, pp)


def _tc_grouped_mlp(block_expert, gx, w_up, w_down, N, D, H, E, NB, P,
                    interpret=False):
    HT = H // NJ

    def body(eids_ref, wg_ref, wl_ref, wd_ref, gx_ref, o_ref,
             gx_s, acc_s, wg_s, wl_s, wd_s):
        j = pl.program_id(0)
        b = pl.program_id(1)
        rows = pl.ds(b * BM, BM)

        @pl.when(b < eids_ref[NB])
        def _():
            new_tile = jnp.logical_or(
                b == 0, eids_ref[b] != eids_ref[jnp.maximum(b - 1, 0)])

            @pl.when(new_tile)
            def _():
                wg_s[...] = wg_ref[0].astype(jnp.bfloat16)
                wl_s[...] = wl_ref[0].astype(jnp.bfloat16)
                wd_s[...] = wd_ref[0].astype(jnp.bfloat16)

            def compute(xb):
                hg = lax.dot_general(xb, wg_s[...], (((1,), (1,)), ((), ())),
                                     preferred_element_type=jnp.float32)
                hl = lax.dot_general(xb, wl_s[...], (((1,), (1,)), ((), ())),
                                     preferred_element_type=jnp.float32)
                g = (hg * jax.nn.sigmoid(hg)) * hl
                return lax.dot_general(g.astype(jnp.bfloat16), wd_s[...],
                                       (((1,), (1,)), ((), ())),
                                       preferred_element_type=jnp.float32)

            @pl.when(j == 0)
            def _():
                xb = gx_ref[...].astype(jnp.bfloat16)
                gx_s[rows, :] = xb
                acc_s[rows, :] = compute(xb).astype(jnp.bfloat16)

            @pl.when(jnp.logical_and(j > 0, j < NJ - 1))
            def _():
                part = compute(gx_s[rows, :])
                acc_s[rows, :] = (acc_s[rows, :].astype(jnp.float32)
                                  + part).astype(jnp.bfloat16)

            @pl.when(j == NJ - 1)
            def _():
                part = compute(gx_s[rows, :])
                o_ref[...] = acc_s[rows, :].astype(jnp.float32) + part

    grid_spec = pltpu.PrefetchScalarGridSpec(
        num_scalar_prefetch=1,
        grid=(NJ, NB),
        in_specs=[
            pl.BlockSpec((1, HT, D), lambda j, b, eids: (eids[b], j, 0)),
            pl.BlockSpec((1, HT, D),
                         lambda j, b, eids: (eids[b], NJ + j, 0)),
            pl.BlockSpec((1, D, HT), lambda j, b, eids: (eids[b], 0, j)),
            pl.BlockSpec((BM, D),
                         lambda j, b, eids: (jnp.where(j == 0, b, 0), 0)),
        ],
        out_specs=pl.BlockSpec(
            (BM, D), lambda j, b, eids: (jnp.where(j == NJ - 1, b, 0), 0)),
        scratch_shapes=[pltpu.VMEM((P, D), jnp.bfloat16),
                        pltpu.VMEM((P, D), jnp.bfloat16),
                        pltpu.VMEM((HT, D), jnp.bfloat16),
                        pltpu.VMEM((HT, D), jnp.bfloat16),
                        pltpu.VMEM((D, HT), jnp.bfloat16)],
    )
    return pl.pallas_call(
        body,
        grid_spec=grid_spec,
        out_shape=jax.ShapeDtypeStruct((P, D), jnp.float32),
        compiler_params=pltpu.CompilerParams(
            dimension_semantics=("arbitrary", "arbitrary"),
            vmem_limit_bytes=100 * 1024 * 1024,
        ),
        interpret=interpret,
    )(block_expert, w_up, w_up, w_down, gx)


def _sc_combine(y, topk_weights, ppN, N, K, D):
    info = plsc.get_sparse_core_info()
    NC, NS = info.num_cores, info.num_subcores
    NW = NC * NS
    per_w = N // NW
    mesh = plsc.VectorSubcoreMesh(core_axis_name="c", subcore_axis_name="s")
    idx0 = ppN[:, 0]
    idx1 = ppN[:, 1]
    twf = topk_weights.reshape(N * K).astype(jnp.float32)

    cp = pltpu.CompilerParams()
    if "needs_layout_passes" in pltpu.CompilerParams.__dataclass_fields__:
        cp = dataclasses.replace(cp, needs_layout_passes=False)

    NCH = per_w // CW

    @functools.partial(
        pl.kernel, mesh=mesh,
        out_type=jax.ShapeDtypeStruct((N, D), jnp.float32),
        compiler_params=cp,
        scratch_types=[pltpu.VMEM((per_w,), jnp.int32),
                       pltpu.VMEM((per_w,), jnp.int32),
                       pltpu.VMEM((per_w * 2,), jnp.float32),
                       pltpu.VMEM((CW, D), jnp.float32),
                       pltpu.VMEM((CW, D), jnp.float32),
                       pltpu.VMEM((CW, D), jnp.float32),
                       pltpu.VMEM((CW, D), jnp.float32),
                       pltpu.SemaphoreType.DMA])
    def combine_kernel(y_hbm, tw_hbm, i0_hbm, i1_hbm, out_hbm,
                       i0_all, i1_all, tw_all, acc0, acc1, rw0, rw1, sem):
        wid = lax.axis_index("s") * NC + lax.axis_index("c")
        base = wid * per_w
        pltpu.sync_copy(i0_hbm.at[pl.ds(base, per_w)], i0_all)
        pltpu.sync_copy(i1_hbm.at[pl.ds(base, per_w)], i1_all)
        pltpu.sync_copy(tw_hbm.at[pl.ds(2 * base, 2 * per_w)], tw_all)
        acc = (acc0, acc1)
        rw = (rw0, rw1)

        def issue(c, buf):
            c0 = pltpu.async_copy(
                y_hbm.at[i0_all.at[pl.ds(c * CW, CW)]], acc[buf], sem)
            c1 = pltpu.async_copy(
                y_hbm.at[i1_all.at[pl.ds(c * CW, CW)]], rw[buf], sem)
            return c0, c1

        cps = [None] * NCH
        cps[0] = issue(0, 0)
        for c in range(NCH):
            cps[c][0].wait()
            cps[c][1].wait()
            if c + 1 < NCH:
                cps[c + 1] = issue(c + 1, (c + 1) % 2)
            av, rv = acc[c % 2], rw[c % 2]

            @pl.loop(0, CW)
            def _(t):
                tw0 = plsc.load_gather(
                    tw_all, [jnp.full((16,), 2 * (c * CW + t), jnp.int32)])
                tw1 = plsc.load_gather(
                    tw_all, [jnp.full((16,), 2 * (c * CW + t) + 1,
                                      jnp.int32)])

                @pl.loop(0, D, step=64)
                def _(d):
                    for u in range(4):
                        sl = pl.ds(d + 16 * u, 16)
                        av[t, sl] = tw0 * av[t, sl] + tw1 * rv[t, sl]

            pltpu.sync_copy(av, out_hbm.at[pl.ds(base + c * CW, CW)])

    return combine_kernel(y, twf, idx0, idx1)


def kernel(hidden_states, topk_weights, topk_ids, w_up, w_down):
    N, D = hidden_states.shape
    K = topk_ids.shape[1]
    E = w_up.shape[0]
    H = w_down.shape[2]
    NB = (N * K) // BM + E
    P = NB * BM

    block_expert, pp = _routing(topk_ids, N, K, E, NB)
    gx = _sc_dispatch(hidden_states, pp, N, K, P, D)
    y = _tc_grouped_mlp(block_expert, gx, w_up, w_down, N, D, H, E, NB, P)
    return _sc_combine(y, topk_weights, pp.reshape(N, K), N, K, D)
